# baseline (speedup 1.0000x reference)
import functools

import jax
import jax.numpy as jnp
from jax import lax
from jax.experimental import pallas as pl
from jax.experimental.pallas import tpu as pltpu
from jax.experimental.pallas import tpu_sc as plsc

MAX_SEQ_LEN = 8192
D_MODEL = 1024
B = 8192

_info = plsc.get_sparse_core_info()
_NC, _NS = _info.num_cores, _info.num_subcores
_NW = _NC * _NS
_BPW = B // _NW
_CHUNK = 32
_NBUF = 3
_NCHUNK = _BPW // _CHUNK


def _gather_body(pe_hbm, pos_hbm, out_hbm, idx_v, *bufs_and_sems):
    rows = bufs_and_sems[:_NBUF]
    gsem = bufs_and_sems[_NBUF:2 * _NBUF]
    wsem = bufs_and_sems[2 * _NBUF:3 * _NBUF]
    wid = lax.axis_index("s") * _NC + lax.axis_index("c")
    base = wid * _BPW

    pltpu.sync_copy(pos_hbm.at[pl.ds(base, _BPW)], idx_v)

    def start_gather(i, b):
        pltpu.async_copy(
            pe_hbm.at[idx_v.at[pl.ds(i * _CHUNK, _CHUNK)]], rows[b], gsem[b]
        )

    for b in range(_NBUF):
        start_gather(b, b)
    writes = {}
    for i in range(_NCHUNK):
        b = i % _NBUF
        pltpu.make_async_copy(
            pe_hbm.at[idx_v.at[pl.ds(i * _CHUNK, _CHUNK)]], rows[b], gsem[b]
        ).wait()
        writes[i] = pltpu.async_copy(
            rows[b], out_hbm.at[pl.ds(base + i * _CHUNK, _CHUNK)], wsem[b]
        )
        nxt = i + _NBUF
        if nxt < _NCHUNK:
            writes[i].wait()
            start_gather(nxt, b)
    for i in range(max(0, _NCHUNK - _NBUF), _NCHUNK):
        writes[i].wait()


@jax.jit
def _gather(pe, positions):
    mesh = plsc.VectorSubcoreMesh(core_axis_name="c", subcore_axis_name="s")
    return pl.kernel(
        _gather_body,
        mesh=mesh,
        out_type=jax.ShapeDtypeStruct((B, D_MODEL), jnp.float32),
        scratch_types=(
            [pltpu.VMEM((_BPW,), jnp.int32)]
            + [pltpu.VMEM((_CHUNK, D_MODEL), jnp.float32) for _ in range(_NBUF)]
            + [pltpu.SemaphoreType.DMA for _ in range(2 * _NBUF)]
        ),
    )(pe, positions)


_TC_BR = 128
_TC_NBLK = B // _TC_BR


def _tc_body(idx_sref, pe_hbm, out_vmem, sem):
    i = pl.program_id(0)
    base = i * _TC_BR
    copies = []
    for j in range(_TC_BR):
        c = pltpu.make_async_copy(
            pe_hbm.at[pl.ds(idx_sref[base + j], 1)], out_vmem.at[pl.ds(j, 1)], sem
        )
        c.start()
        copies.append(c)
    for c in copies:
        c.wait()


@jax.jit
def _tc_gather(pe, positions):
    grid_spec = pltpu.PrefetchScalarGridSpec(
        num_scalar_prefetch=1,
        grid=(_TC_NBLK,),
        in_specs=[pl.BlockSpec(memory_space=pl.ANY)],
        out_specs=pl.BlockSpec((_TC_BR, D_MODEL), lambda i, idx: (i, 0)),
        scratch_shapes=[pltpu.SemaphoreType.DMA],
    )
    return pl.pallas_call(
        _tc_body,
        grid_spec=grid_spec,
        out_shape=jax.ShapeDtypeStruct((B, D_MODEL), jnp.float32),
    )(positions, pe)


def kernel(pe, positions):
    return _tc_gather(pe, positions.astype(jnp.int32))


# speedup vs baseline: 2.3243x; 2.3243x over previous
import functools

import jax
import jax.numpy as jnp
from jax import lax
from jax.experimental import pallas as pl
from jax.experimental.pallas import tpu as pltpu
from jax.experimental.pallas import tpu_sc as plsc

MAX_SEQ_LEN = 8192
D_MODEL = 1024
B = 8192

_info = plsc.get_sparse_core_info()
_NC, _NS = _info.num_cores, _info.num_subcores
_NW = _NC * _NS
_BPW = B // _NW
_CHUNK = 16
_NBUF = 6
_NCHUNK = _BPW // _CHUNK


def _gather_body(pe_hbm, pos_hbm, out_hbm, idx_v, *bufs_and_sems):
    rows = bufs_and_sems[:_NBUF]
    gsem = bufs_and_sems[_NBUF:2 * _NBUF]
    wsem = bufs_and_sems[2 * _NBUF:3 * _NBUF]
    wid = lax.axis_index("s") * _NC + lax.axis_index("c")
    base = wid * _BPW

    pltpu.sync_copy(pos_hbm.at[pl.ds(base, _BPW)], idx_v)

    def start_gather(i, b):
        pltpu.async_copy(
            pe_hbm.at[idx_v.at[pl.ds(i * _CHUNK, _CHUNK)]], rows[b], gsem[b]
        )

    for b in range(_NBUF):
        start_gather(b, b)
    writes = {}
    for i in range(_NCHUNK):
        b = i % _NBUF
        pltpu.make_async_copy(
            pe_hbm.at[idx_v.at[pl.ds(i * _CHUNK, _CHUNK)]], rows[b], gsem[b]
        ).wait()
        writes[i] = pltpu.async_copy(
            rows[b], out_hbm.at[pl.ds(base + i * _CHUNK, _CHUNK)], wsem[b]
        )
        nxt = i + _NBUF
        if nxt < _NCHUNK:
            writes[i].wait()
            start_gather(nxt, b)
    for i in range(max(0, _NCHUNK - _NBUF), _NCHUNK):
        writes[i].wait()


@jax.jit
def _gather(pe, positions):
    mesh = plsc.VectorSubcoreMesh(core_axis_name="c", subcore_axis_name="s")
    return pl.kernel(
        _gather_body,
        mesh=mesh,
        out_type=jax.ShapeDtypeStruct((B, D_MODEL), jnp.float32),
        scratch_types=(
            [pltpu.VMEM((_BPW,), jnp.int32)]
            + [pltpu.VMEM((_CHUNK, D_MODEL), jnp.float32) for _ in range(_NBUF)]
            + [pltpu.SemaphoreType.DMA for _ in range(2 * _NBUF)]
        ),
    )(pe, positions)


def kernel(pe, positions):
    return _gather(pe, positions.astype(jnp.int32))
